# Initial kernel scaffold; baseline (speedup 1.0000x reference)
import jax
import jax.numpy as jnp
from jax.experimental import pallas as pl


def kernel(boxes, scores, labels):
    raise NotImplementedError("write your pallas kernel here")


# speedup vs baseline: 1640.1274x; 1640.1274x over previous
import functools

import jax
import jax.numpy as jnp
from jax import lax
from jax.experimental import pallas as pl

_VOTE_THRESH = 0.65
_MAX_DET = 100


def _nms_body(x1_ref, y1_ref, x2_ref, y2_ref, sc_ref, lab_ref, out_ref, *, n):
    shape = x1_ref.shape
    x1 = x1_ref[...]
    y1 = y1_ref[...]
    x2 = x2_ref[...]
    y2 = y2_ref[...]
    sc = sc_ref[...]
    lab = lab_ref[...]

    mc = jnp.max(jnp.maximum(x2, y2)) + 1.0

    ox1 = x1 + lab * mc
    oy1 = y1 + lab * mc
    ox2 = x2 + lab * mc
    oy2 = y2 + lab * mc
    area = (ox2 - ox1) * (oy2 - oy1)

    lin = (lax.broadcasted_iota(jnp.int32, shape, 0) * shape[1]
           + lax.broadcasted_iota(jnp.int32, shape, 1))
    lane = lax.broadcasted_iota(jnp.int32, (1, 128), 1)
    big = jnp.int32(shape[0] * shape[1] + 1)

    zrow = jnp.zeros((1, 128), jnp.float32)

    def step(k, carry):
        unf, asx1, asy1, asx2, asy2, asw, assc, aslab, asval = carry
        un = unf > 0.0
        ms = jnp.where(un, sc, -1.0)
        m = jnp.max(ms)
        any_left = m >= 0.0
        sel = un & (ms == m)
        head = jnp.min(jnp.where(sel, lin, big))
        hsel = lin == head
        hx1 = jnp.sum(jnp.where(hsel, ox1, 0.0))
        hy1 = jnp.sum(jnp.where(hsel, oy1, 0.0))
        hx2 = jnp.sum(jnp.where(hsel, ox2, 0.0))
        hy2 = jnp.sum(jnp.where(hsel, oy2, 0.0))
        harea = (hx2 - hx1) * (hy2 - hy1)
        w = jnp.maximum(jnp.minimum(hx2, ox2) - jnp.maximum(hx1, ox1), 0.0)
        h = jnp.maximum(jnp.minimum(hy2, oy2) - jnp.maximum(hy1, oy1), 0.0)
        inter = w * h
        iou = inter / (harea + area - inter)
        merge = (iou >= _VOTE_THRESH) & un & any_left
        mw = jnp.where(merge, sc, 0.0)
        km = lane == k
        asx1 = jnp.where(km, jnp.sum(mw * ox1), asx1)
        asy1 = jnp.where(km, jnp.sum(mw * oy1), asy1)
        asx2 = jnp.where(km, jnp.sum(mw * ox2), asx2)
        asy2 = jnp.where(km, jnp.sum(mw * oy2), asy2)
        asw = jnp.where(km, jnp.sum(mw), asw)
        assc = jnp.where(km, m, assc)
        aslab = jnp.where(km, jnp.max(jnp.where(merge, lab, 0.0)), aslab)
        asval = jnp.where(km & any_left, 1.0, asval)
        unf = jnp.where(merge, 0.0, unf)
        return (unf, asx1, asy1, asx2, asy2, asw, assc, aslab, asval)

    un0 = jnp.where(lin < n, 1.0, 0.0)
    init = (un0, zrow, zrow, zrow, zrow, zrow, zrow, zrow, zrow)
    carry = lax.fori_loop(0, _MAX_DET, step, init)
    _, asx1, asy1, asx2, asy2, asw, assc, aslab, asval = carry

    v = asval > 0.0
    denom = jnp.where(v, asw, 1.0)
    off = aslab * mc
    r0 = jnp.where(v, asx1 / denom - off, 0.0)
    r1 = jnp.where(v, asy1 / denom - off, 0.0)
    r2 = jnp.where(v, asx2 / denom - off, 0.0)
    r3 = jnp.where(v, asy2 / denom - off, 0.0)
    r4 = jnp.where(v, assc, 0.0)
    r5 = jnp.where(v, aslab, -1.0)
    out_ref[...] = jnp.concatenate([r0, r1, r2, r3, r4, r5, zrow, zrow], axis=0)


def _prep(boxes, scores, labels):
    n = boxes.shape[0]
    rows = -(-n // 128)
    rows = -(-rows // 8) * 8
    p = rows * 128 - n
    labf = labels.astype(jnp.float32)

    def pad(a):
        return jnp.pad(a, (0, p)).reshape(rows, 128)

    return (pad(boxes[:, 0]), pad(boxes[:, 1]), pad(boxes[:, 2]),
            pad(boxes[:, 3]), pad(scores), pad(labf)), n


def kernel(boxes, scores, labels):
    args, n = _prep(boxes, scores, labels)
    out = pl.pallas_call(
        functools.partial(_nms_body, n=n),
        out_shape=jax.ShapeDtypeStruct((8, 128), jnp.float32),
    )(*args)
    out_boxes = out[0:4, :_MAX_DET].T
    out_scores = out[4, :_MAX_DET]
    out_labels = out[5, :_MAX_DET]
    return out_boxes, out_scores, out_labels
